# baseline (speedup 1.0000x reference)
import functools

import jax
import jax.numpy as jnp
from jax import lax
from jax.experimental import pallas as pl
from jax.experimental.pallas import tpu as pltpu
from jax.experimental.pallas import tpu_sc as plsc

B = 4096
N = 204800
EMB = 64
HID = 128

NC = 2
NS = 16
NW = NC * NS
ROWS_W = N // NW
CH = 128
NCH = ROWS_W // CH
NCHUNKS = N // CH
CH_SC = NCHUNKS // NS

_mesh = plsc.VectorSubcoreMesh(core_axis_name="c", subcore_axis_name="s")


@functools.partial(
    pl.kernel,
    mesh=_mesh,
    out_type=jax.ShapeDtypeStruct((N, HID), jnp.float32),
    scratch_types=[
        pltpu.VMEM((NCH, CH), jnp.int32),
        pltpu.VMEM((4, CH, EMB), jnp.float32),
        pltpu.SemaphoreType.DMA((4,)),
        pltpu.SemaphoreType.DMA((4,)),
    ],
    compiler_params=pltpu.CompilerParams(use_tc_tiling_on_sc=False),
)
def _gather_rep(emb_hbm, idx_hbm, out_hbm, idx_v, buf_v, gsems, osems):
    c = lax.axis_index("c")
    s = lax.axis_index("s")
    wid = s * NC + c
    base = wid * ROWS_W
    pltpu.sync_copy(idx_hbm.at[wid], idx_v)

    def gat(j, sl):
        return pltpu.make_async_copy(
            emb_hbm.at[idx_v.at[j]], buf_v.at[sl], gsems.at[sl])

    def out(j, sl):
        return pltpu.make_async_copy(
            buf_v.at[sl],
            out_hbm.at[pl.ds(base + j * CH, CH), pl.ds(0, EMB)],
            osems.at[sl])

    for j in range(3):
        gat(j, j).start()

    def body(j, carry):
        sl = lax.rem(j, 4)
        gat(j, sl).wait()
        out(j, sl).start()

        @pl.when(j + 3 < NCH)
        def _():
            sl3 = lax.rem(j + 3, 4)

            @pl.when(j >= 1)
            def _():
                out(j - 1, sl3).wait()

            gat(j + 3, sl3).start()

        return carry

    lax.fori_loop(0, NCH, body, 0)
    for j in range(NCH - 4, NCH):
        out(j, j % 4).wait()


TILE = 2048


def _mlp_body(xs_ref, gcT_ref, w0a_ref, w0b_ref, b0_ref, w1_ref, b1_ref,
              wf_ref, bf_ref, eye_ref, xs_out, ex_ref):
    f32 = jnp.float32
    xr = xs_ref[:, :EMB]
    gcT = gcT_ref[...]
    gc = jax.lax.dot_general(gcT, eye_ref[...], (((0,), (0,)), ((), ())),
                             precision=jax.lax.Precision.HIGHEST,
                             preferred_element_type=f32)
    xs_out[:, :EMB] = xr
    xs_out[:, EMB:] = gc
    h = jnp.dot(xr, w0a_ref[...], preferred_element_type=f32)
    h += jax.lax.dot_general(gcT, w0b_ref[...], (((0,), (0,)), ((), ())),
                             preferred_element_type=f32)
    h = jnp.maximum(h + b0_ref[...], 0.0)
    h = jnp.dot(h, w1_ref[...], preferred_element_type=f32)
    h = jnp.maximum(h + b1_ref[...], 0.0)
    logits = jnp.sum(h * wf_ref[...], axis=1) + bf_ref[0, 0]
    ex_ref[...] = jnp.exp(logits).reshape(TILE // CH, CH)


_mlp = pl.pallas_call(
    _mlp_body,
    grid=(N // TILE,),
    in_specs=[
        pl.BlockSpec((TILE, HID), lambda i: (i, 0)),
        pl.BlockSpec((EMB, TILE), lambda i: (0, i)),
        pl.BlockSpec((EMB, HID), lambda i: (0, 0)),
        pl.BlockSpec((EMB, HID), lambda i: (0, 0)),
        pl.BlockSpec((1, HID), lambda i: (0, 0)),
        pl.BlockSpec((HID, HID), lambda i: (0, 0)),
        pl.BlockSpec((1, HID), lambda i: (0, 0)),
        pl.BlockSpec((1, HID), lambda i: (0, 0)),
        pl.BlockSpec((1, 1), lambda i: (0, 0)),
        pl.BlockSpec((EMB, EMB), lambda i: (0, 0)),
    ],
    out_specs=[
        pl.BlockSpec((TILE, HID), lambda i: (i, 0)),
        pl.BlockSpec((TILE // CH, CH), lambda i: (i, 0)),
    ],
    out_shape=[
        jax.ShapeDtypeStruct((N, HID), jnp.float32),
        jax.ShapeDtypeStruct((NCHUNKS, CH), jnp.float32),
    ],
    input_output_aliases={0: 0},
)


@functools.partial(
    pl.kernel,
    mesh=_mesh,
    out_type=jax.ShapeDtypeStruct((NW, NCH, CH), jnp.float32),
    scratch_types=[
        pltpu.VMEM((CH_SC, CH), jnp.int32),
        pltpu.VMEM((CH_SC, CH), jnp.float32),
        pltpu.VMEM((NCH, CH), jnp.int32),
        pltpu.VMEM((NCH, CH), jnp.float32),
        pltpu.VMEM((NCH, CH), jnp.float32),
        pltpu.VMEM((B,), jnp.float32),
        pltpu.VMEM((B,), jnp.float32),
        pltpu.VMEM_SHARED((NS * B,), jnp.float32),
    ],
    compiler_params=pltpu.CompilerParams(needs_layout_passes=False),
)
def _seg_softmax(ex_sc_hbm, idx_sc_hbm, ex_hbm, idx_hbm, out_hbm,
                 idx_a, ex_a, idx_b, ex_b, out_v, table_v, stage_v, table_sh):
    c = lax.axis_index("c")
    s = lax.axis_index("s")
    wid = s * NC + c

    def zbody(i, carry):
        table_v[pl.ds(i * 16, 16)] = jnp.zeros((16,), jnp.float32)
        return carry
    lax.fori_loop(0, B // 16, zbody, 0)
    pltpu.sync_copy(table_v, table_sh.at[pl.ds(s * B, B)])

    pltpu.sync_copy(idx_sc_hbm.at[s], idx_a)
    pltpu.sync_copy(ex_sc_hbm.at[s], ex_a)
    off = (s * B).astype(jnp.int32)

    def obody(i, carry):
        r = i // (CH // 16)
        k = i % (CH // 16)
        sl = pl.ds(k * 16, 16)
        idx_a[r, sl] = idx_a[r, sl] + off
        return carry

    lax.fori_loop(0, CH_SC * (CH // 16), obody, 0)

    def sbody(j, carry):
        pltpu.sync_copy(ex_a.at[j], table_sh.at[idx_a.at[j]], add=True)
        return carry

    lax.fori_loop(0, CH_SC, sbody, 0)
    plsc.subcore_barrier()

    pltpu.sync_copy(table_sh.at[pl.ds(0, B)], table_v)

    def cbody(r, carry):
        pltpu.sync_copy(table_sh.at[pl.ds(r * B, B)], stage_v)

        def abody(i, carry2):
            sl = pl.ds(i * 16, 16)
            table_v[sl] = table_v[sl] + stage_v[sl]
            return carry2
        lax.fori_loop(0, B // 16, abody, 0)
        return carry

    lax.fori_loop(1, NS, cbody, 0)

    pltpu.sync_copy(idx_hbm.at[wid], idx_b)
    pltpu.sync_copy(ex_hbm.at[wid], ex_b)

    def dbody(j, carry):
        def inner(k, carry2):
            idx16 = idx_b[j, pl.ds(k * 16, 16)]
            ex16 = ex_b[j, pl.ds(k * 16, 16)]
            den16 = plsc.load_gather(table_v, [idx16])
            out_v[j, pl.ds(k * 16, 16)] = ex16 / den16
            return carry2
        return lax.fori_loop(0, CH // 16, inner, carry)

    lax.fori_loop(0, NCH, dbody, 0)
    pltpu.sync_copy(out_v, out_hbm.at[wid])


def kernel(g_emb, g_candidates_emb, batch_idx, W0, b0, W1, b1, Wf, bf):
    idx3 = batch_idx.reshape(NW, NCH, CH)
    idx_sc = batch_idx.reshape(NS, CH_SC, CH)
    xs0 = _gather_rep(g_emb, idx3)
    x_states, ex = _mlp(
        xs0, g_candidates_emb.T,
        W0[:EMB], W0[EMB:],
        b0.reshape(1, HID), W1, b1.reshape(1, HID),
        Wf.reshape(1, HID), bf.reshape(1, 1),
        jnp.eye(EMB, dtype=jnp.float32),
    )
    probs = _seg_softmax(ex.reshape(NS, CH_SC, CH), idx_sc,
                         ex.reshape(NW, NCH, CH), idx3)
    return (g_emb, x_states, probs.reshape(N))


# speedup vs baseline: 1.9414x; 1.9414x over previous
import functools

import jax
import jax.numpy as jnp
from jax import lax
from jax.experimental import pallas as pl
from jax.experimental.pallas import tpu as pltpu
from jax.experimental.pallas import tpu_sc as plsc

B = 4096
N = 204800
EMB = 64
HID = 128

NC = 2
NS = 16
NW = NC * NS
ROWS_W = N // NW
CH = 128
NCH = ROWS_W // CH
NCHUNKS = N // CH
CH_SC = NCHUNKS // NS

_mesh = plsc.VectorSubcoreMesh(core_axis_name="c", subcore_axis_name="s")


@functools.partial(
    pl.kernel,
    mesh=_mesh,
    out_type=jax.ShapeDtypeStruct((N, HID), jnp.float32),
    scratch_types=[
        pltpu.VMEM((NCH, CH), jnp.int32),
        pltpu.VMEM((4, CH, EMB), jnp.float32),
        pltpu.SemaphoreType.DMA((4,)),
        pltpu.SemaphoreType.DMA((4,)),
    ],
    compiler_params=pltpu.CompilerParams(use_tc_tiling_on_sc=False),
)
def _gather_rep(emb_hbm, idx_hbm, out_hbm, idx_v, buf_v, gsems, osems):
    c = lax.axis_index("c")
    s = lax.axis_index("s")
    wid = s * NC + c
    base = wid * ROWS_W
    pltpu.sync_copy(idx_hbm.at[wid], idx_v)

    def gat(j, sl):
        return pltpu.make_async_copy(
            emb_hbm.at[idx_v.at[j]], buf_v.at[sl], gsems.at[sl])

    def out(j, sl):
        return pltpu.make_async_copy(
            buf_v.at[sl],
            out_hbm.at[pl.ds(base + j * CH, CH), pl.ds(0, EMB)],
            osems.at[sl])

    for j in range(3):
        gat(j, j).start()

    def body(j, carry):
        sl = lax.rem(j, 4)
        gat(j, sl).wait()
        out(j, sl).start()

        @pl.when(j + 3 < NCH)
        def _():
            sl3 = lax.rem(j + 3, 4)

            @pl.when(j >= 1)
            def _():
                out(j - 1, sl3).wait()

            gat(j + 3, sl3).start()

        return carry

    lax.fori_loop(0, NCH, body, 0)
    for j in range(NCH - 4, NCH):
        out(j, j % 4).wait()


TILE = 2048


def _mlp_body(xs_ref, gcT_ref, w0a_ref, w0b_ref, b0_ref, w1_ref, b1_ref,
              wf_ref, bf_ref, eye_ref, xs_out, ex_ref):
    f32 = jnp.float32
    xr = xs_ref[:, :EMB]
    gcT = gcT_ref[...]
    gc = jax.lax.dot_general(gcT, eye_ref[...], (((0,), (0,)), ((), ())),
                             preferred_element_type=f32)
    xs_out[:, :EMB] = xr
    xs_out[:, EMB:] = gc
    h = jnp.dot(xr, w0a_ref[...], preferred_element_type=f32)
    h += jax.lax.dot_general(gcT, w0b_ref[...], (((0,), (0,)), ((), ())),
                             preferred_element_type=f32)
    h = jnp.maximum(h + b0_ref[...], 0.0)
    h = jnp.dot(h, w1_ref[...], preferred_element_type=f32)
    h = jnp.maximum(h + b1_ref[...], 0.0)
    logits = jnp.sum(h * wf_ref[...], axis=1) + bf_ref[0, 0]
    ex_ref[...] = jnp.exp(logits).reshape(TILE // CH, CH)


_mlp = pl.pallas_call(
    _mlp_body,
    grid=(N // TILE,),
    in_specs=[
        pl.BlockSpec((TILE, HID), lambda i: (i, 0)),
        pl.BlockSpec((EMB, TILE), lambda i: (0, i)),
        pl.BlockSpec((EMB, HID), lambda i: (0, 0)),
        pl.BlockSpec((EMB, HID), lambda i: (0, 0)),
        pl.BlockSpec((1, HID), lambda i: (0, 0)),
        pl.BlockSpec((HID, HID), lambda i: (0, 0)),
        pl.BlockSpec((1, HID), lambda i: (0, 0)),
        pl.BlockSpec((1, HID), lambda i: (0, 0)),
        pl.BlockSpec((1, 1), lambda i: (0, 0)),
        pl.BlockSpec((EMB, EMB), lambda i: (0, 0)),
    ],
    out_specs=[
        pl.BlockSpec((TILE, HID), lambda i: (i, 0)),
        pl.BlockSpec((TILE // CH, CH), lambda i: (i, 0)),
    ],
    out_shape=[
        jax.ShapeDtypeStruct((N, HID), jnp.float32),
        jax.ShapeDtypeStruct((NCHUNKS, CH), jnp.float32),
    ],
    input_output_aliases={0: 0},
)


@functools.partial(
    pl.kernel,
    mesh=_mesh,
    out_type=jax.ShapeDtypeStruct((NW, NCH, CH), jnp.float32),
    scratch_types=[
        pltpu.VMEM((CH_SC, CH), jnp.int32),
        pltpu.VMEM((CH_SC, CH), jnp.float32),
        pltpu.VMEM((NCH, CH), jnp.int32),
        pltpu.VMEM((NCH, CH), jnp.float32),
        pltpu.VMEM((NCH, CH), jnp.float32),
        pltpu.VMEM((B,), jnp.float32),
        pltpu.VMEM((B,), jnp.float32),
        pltpu.VMEM_SHARED((NS * B,), jnp.float32),
    ],
    compiler_params=pltpu.CompilerParams(needs_layout_passes=False),
)
def _seg_softmax(ex_sc_hbm, idx_sc_hbm, ex_hbm, idx_hbm, out_hbm,
                 idx_a, ex_a, idx_b, ex_b, out_v, table_v, stage_v, table_sh):
    c = lax.axis_index("c")
    s = lax.axis_index("s")
    wid = s * NC + c

    def zbody(i, carry):
        table_v[pl.ds(i * 16, 16)] = jnp.zeros((16,), jnp.float32)
        return carry
    lax.fori_loop(0, B // 16, zbody, 0)
    pltpu.sync_copy(table_v, table_sh.at[pl.ds(s * B, B)])

    pltpu.sync_copy(idx_sc_hbm.at[s], idx_a)
    pltpu.sync_copy(ex_sc_hbm.at[s], ex_a)
    off = (s * B).astype(jnp.int32)

    def obody(i, carry):
        r = i // (CH // 16)
        k = i % (CH // 16)
        sl = pl.ds(k * 16, 16)
        idx_a[r, sl] = idx_a[r, sl] + off
        return carry

    lax.fori_loop(0, CH_SC * (CH // 16), obody, 0)

    def sbody(j, carry):
        pltpu.sync_copy(ex_a.at[j], table_sh.at[idx_a.at[j]], add=True)
        return carry

    lax.fori_loop(0, CH_SC, sbody, 0)
    plsc.subcore_barrier()

    pltpu.sync_copy(table_sh.at[pl.ds(0, B)], table_v)

    def cbody(r, carry):
        pltpu.sync_copy(table_sh.at[pl.ds(r * B, B)], stage_v)

        def abody(i, carry2):
            sl = pl.ds(i * 16, 16)
            table_v[sl] = table_v[sl] + stage_v[sl]
            return carry2
        lax.fori_loop(0, B // 16, abody, 0)
        return carry

    lax.fori_loop(1, NS, cbody, 0)

    pltpu.sync_copy(idx_hbm.at[wid], idx_b)
    pltpu.sync_copy(ex_hbm.at[wid], ex_b)

    def dbody(j, carry):
        def inner(k, carry2):
            idx16 = idx_b[j, pl.ds(k * 16, 16)]
            ex16 = ex_b[j, pl.ds(k * 16, 16)]
            den16 = plsc.load_gather(table_v, [idx16])
            out_v[j, pl.ds(k * 16, 16)] = ex16 / den16
            return carry2
        return lax.fori_loop(0, CH // 16, inner, carry)

    lax.fori_loop(0, NCH, dbody, 0)
    pltpu.sync_copy(out_v, out_hbm.at[wid])


def kernel(g_emb, g_candidates_emb, batch_idx, W0, b0, W1, b1, Wf, bf):
    idx3 = batch_idx.reshape(NW, NCH, CH)
    idx_sc = batch_idx.reshape(NS, CH_SC, CH)
    xs0 = _gather_rep(g_emb, idx3)
    x_states, ex = _mlp(
        xs0, g_candidates_emb.T,
        W0[:EMB], W0[EMB:],
        b0.reshape(1, HID), W1, b1.reshape(1, HID),
        Wf.reshape(1, HID), bf.reshape(1, 1),
        jnp.eye(EMB, dtype=jnp.float32),
    )
    probs = _seg_softmax(ex.reshape(NS, CH_SC, CH), idx_sc,
                         ex.reshape(NW, NCH, CH), idx3)
    return (g_emb, x_states, probs.reshape(N))


# speedup vs baseline: 2.0334x; 1.0474x over previous
import functools

import jax
import jax.numpy as jnp
from jax import lax
from jax.experimental import pallas as pl
from jax.experimental.pallas import tpu as pltpu
from jax.experimental.pallas import tpu_sc as plsc

B = 4096
N = 204800
EMB = 64
HID = 128

NC = 2
NS = 16
NW = NC * NS
ROWS_W = N // NW
CH = 128
NCH = ROWS_W // CH
NCHUNKS = N // CH
CH_SC = NCHUNKS // NS

_mesh = plsc.VectorSubcoreMesh(core_axis_name="c", subcore_axis_name="s")


@functools.partial(
    pl.kernel,
    mesh=_mesh,
    out_type=jax.ShapeDtypeStruct((N, HID), jnp.float32),
    scratch_types=[
        pltpu.VMEM((NCH, CH), jnp.int32),
        pltpu.VMEM((4, CH, EMB), jnp.float32),
        pltpu.SemaphoreType.DMA((4,)),
        pltpu.SemaphoreType.DMA((4,)),
    ],
    compiler_params=pltpu.CompilerParams(use_tc_tiling_on_sc=False),
)
def _gather_rep(emb_hbm, idx_hbm, out_hbm, idx_v, buf_v, gsems, osems):
    c = lax.axis_index("c")
    s = lax.axis_index("s")
    wid = s * NC + c
    base = wid * ROWS_W
    pltpu.sync_copy(idx_hbm.at[wid], idx_v)

    def gat(j, sl):
        return pltpu.make_async_copy(
            emb_hbm.at[idx_v.at[j]], buf_v.at[sl], gsems.at[sl])

    def out(j, sl):
        return pltpu.make_async_copy(
            buf_v.at[sl],
            out_hbm.at[pl.ds(base + j * CH, CH), pl.ds(0, EMB)],
            osems.at[sl])

    for j in range(3):
        gat(j, j).start()

    def body(j, carry):
        sl = lax.rem(j, 4)
        gat(j, sl).wait()
        out(j, sl).start()

        @pl.when(j + 3 < NCH)
        def _():
            sl3 = lax.rem(j + 3, 4)

            @pl.when(j >= 1)
            def _():
                out(j - 1, sl3).wait()

            gat(j + 3, sl3).start()

        return carry

    lax.fori_loop(0, NCH, body, 0)
    for j in range(NCH - 4, NCH):
        out(j, j % 4).wait()


TILE = 2048


def _mlp_body(xs_ref, gcT_ref, w0a_ref, w0b_ref, b0_ref, w1_ref, b1_ref,
              wf_ref, bf_ref, eye_ref, xs_out, ex_ref):
    f32 = jnp.float32
    xr = xs_ref[:, :EMB]
    gcT = gcT_ref[...]
    gc = jax.lax.dot_general(gcT, eye_ref[...], (((0,), (0,)), ((), ())),
                             preferred_element_type=f32)
    xs_out[:, :EMB] = xr
    xs_out[:, EMB:] = gc
    h = jnp.dot(xr, w0a_ref[...], preferred_element_type=f32)
    h += jax.lax.dot_general(gcT, w0b_ref[...], (((0,), (0,)), ((), ())),
                             preferred_element_type=f32)
    h = jnp.maximum(h + b0_ref[...], 0.0)
    h = jnp.dot(h, w1_ref[...], preferred_element_type=f32)
    h = jnp.maximum(h + b1_ref[...], 0.0)
    logits = jnp.sum(h * wf_ref[...], axis=1) + bf_ref[0, 0]
    ex_ref[...] = jnp.exp(logits).reshape(TILE // CH, CH)


_mlp = pl.pallas_call(
    _mlp_body,
    grid=(N // TILE,),
    in_specs=[
        pl.BlockSpec((TILE, HID), lambda i: (i, 0)),
        pl.BlockSpec((EMB, TILE), lambda i: (0, i)),
        pl.BlockSpec((EMB, HID), lambda i: (0, 0)),
        pl.BlockSpec((EMB, HID), lambda i: (0, 0)),
        pl.BlockSpec((1, HID), lambda i: (0, 0)),
        pl.BlockSpec((HID, HID), lambda i: (0, 0)),
        pl.BlockSpec((1, HID), lambda i: (0, 0)),
        pl.BlockSpec((1, HID), lambda i: (0, 0)),
        pl.BlockSpec((1, 1), lambda i: (0, 0)),
        pl.BlockSpec((EMB, EMB), lambda i: (0, 0)),
    ],
    out_specs=[
        pl.BlockSpec((TILE, HID), lambda i: (i, 0)),
        pl.BlockSpec((TILE // CH, CH), lambda i: (i, 0)),
    ],
    out_shape=[
        jax.ShapeDtypeStruct((N, HID), jnp.float32),
        jax.ShapeDtypeStruct((NCHUNKS, CH), jnp.float32),
    ],
    input_output_aliases={0: 0},
)


@functools.partial(
    pl.kernel,
    mesh=_mesh,
    out_type=jax.ShapeDtypeStruct((NW, NCH, CH), jnp.float32),
    scratch_types=[
        pltpu.VMEM((CH_SC, CH), jnp.int32),
        pltpu.VMEM((CH_SC, CH), jnp.float32),
        pltpu.VMEM((NCH, CH), jnp.int32),
        pltpu.VMEM((NCH, CH), jnp.float32),
        pltpu.VMEM((NCH, CH), jnp.float32),
        pltpu.VMEM((B,), jnp.float32),
        pltpu.VMEM((B,), jnp.float32),
        pltpu.VMEM_SHARED((NS * B,), jnp.float32),
    ],
    compiler_params=pltpu.CompilerParams(needs_layout_passes=False),
)
def _seg_softmax(ex_sc_hbm, idx_sc_hbm, ex_hbm, idx_hbm, out_hbm,
                 idx_a, ex_a, idx_b, ex_b, out_v, table_v, stage_v, table_sh):
    c = lax.axis_index("c")
    s = lax.axis_index("s")
    wid = s * NC + c

    def zbody(i, carry):
        table_v[pl.ds(i * 16, 16)] = jnp.zeros((16,), jnp.float32)
        return carry
    lax.fori_loop(0, B // 16, zbody, 0)
    pltpu.sync_copy(table_v, table_sh.at[pl.ds(s * B, B)])

    pltpu.sync_copy(idx_sc_hbm.at[s], idx_a)
    pltpu.sync_copy(ex_sc_hbm.at[s], ex_a)
    off = (s * B).astype(jnp.int32)

    def obody(i, carry):
        r = i // (CH // 16)
        k = i % (CH // 16)
        sl = pl.ds(k * 16, 16)
        idx_a[r, sl] = idx_a[r, sl] + off
        return carry

    lax.fori_loop(0, CH_SC * (CH // 16), obody, 0)

    def sbody(j, carry):
        pltpu.sync_copy(ex_a.at[j], table_sh.at[idx_a.at[j]], add=True)
        return carry

    pltpu.sync_copy(idx_hbm.at[wid], idx_b)
    pltpu.sync_copy(ex_hbm.at[wid], ex_b)

    lax.fori_loop(0, CH_SC, sbody, 0)
    plsc.subcore_barrier()

    lo = idx_b[0, pl.ds(0, 16)][0]
    hi = idx_b[NCH - 1, pl.ds(CH - 16, 16)][15]
    BLK = 256
    kb0 = lo // BLK
    nb = hi // BLK - kb0 + 1

    def cpy_body(k, carry):
        base_b = (kb0 + k) * BLK
        pltpu.sync_copy(table_sh.at[pl.ds(base_b, BLK)],
                        table_v.at[pl.ds(base_b, BLK)])
        return carry

    lax.fori_loop(0, nb, cpy_body, 0)

    def cbody(r, carry):
        def kbody(k, carry2):
            base_b = (kb0 + k) * BLK
            pltpu.sync_copy(table_sh.at[pl.ds(r * B + base_b, BLK)],
                            stage_v.at[pl.ds(0, BLK)])

            def abody(i, carry3):
                dst = pl.ds(base_b + i * 16, 16)
                table_v[dst] = table_v[dst] + stage_v[pl.ds(i * 16, 16)]
                return carry3
            return lax.fori_loop(0, BLK // 16, abody, carry2)
        return lax.fori_loop(0, nb, kbody, carry)

    lax.fori_loop(1, NS, cbody, 0)


    def dbody(j, carry):
        def inner(k, carry2):
            idx16 = idx_b[j, pl.ds(k * 16, 16)]
            ex16 = ex_b[j, pl.ds(k * 16, 16)]
            den16 = plsc.load_gather(table_v, [idx16])
            out_v[j, pl.ds(k * 16, 16)] = ex16 / den16
            return carry2
        return lax.fori_loop(0, CH // 16, inner, carry)

    lax.fori_loop(0, NCH, dbody, 0)
    pltpu.sync_copy(out_v, out_hbm.at[wid])


def kernel(g_emb, g_candidates_emb, batch_idx, W0, b0, W1, b1, Wf, bf):
    idx3 = batch_idx.reshape(NW, NCH, CH)
    idx_sc = batch_idx.reshape(NS, CH_SC, CH)
    xs0 = _gather_rep(g_emb, idx3)
    x_states, ex = _mlp(
        xs0, g_candidates_emb.T,
        W0[:EMB], W0[EMB:],
        b0.reshape(1, HID), W1, b1.reshape(1, HID),
        Wf.reshape(1, HID), bf.reshape(1, 1),
        jnp.eye(EMB, dtype=jnp.float32),
    )
    probs = _seg_softmax(ex.reshape(NS, CH_SC, CH), idx_sc,
                         ex.reshape(NW, NCH, CH), idx3)
    return (g_emb, x_states, probs.reshape(N))


# speedup vs baseline: 2.2507x; 1.1069x over previous
import functools

import jax
import jax.numpy as jnp
from jax import lax
from jax.experimental import pallas as pl
from jax.experimental.pallas import tpu as pltpu
from jax.experimental.pallas import tpu_sc as plsc

B = 4096
N = 204800
EMB = 64
HID = 128

NC = 2
NS = 16
NW = NC * NS
ROWS_W = N // NW
CH = 128
NCH = ROWS_W // CH
NCHUNKS = N // CH
CH_SC = NCHUNKS // NS

_mesh = plsc.VectorSubcoreMesh(core_axis_name="c", subcore_axis_name="s")


@functools.partial(
    pl.kernel,
    mesh=_mesh,
    out_type=jax.ShapeDtypeStruct((N, HID), jnp.float32),
    scratch_types=[
        pltpu.VMEM((NCH, CH), jnp.int32),
        pltpu.VMEM((4, CH, EMB), jnp.float32),
        pltpu.SemaphoreType.DMA((4,)),
        pltpu.SemaphoreType.DMA((4,)),
    ],
    compiler_params=pltpu.CompilerParams(use_tc_tiling_on_sc=False),
)
def _gather_rep(emb_hbm, idx_hbm, out_hbm, idx_v, buf_v, gsems, osems):
    c = lax.axis_index("c")
    s = lax.axis_index("s")
    wid = s * NC + c
    base = wid * ROWS_W
    pltpu.sync_copy(idx_hbm.at[wid], idx_v)

    def gat(j, sl):
        return pltpu.make_async_copy(
            emb_hbm.at[idx_v.at[j]], buf_v.at[sl], gsems.at[sl])

    def out(j, sl):
        return pltpu.make_async_copy(
            buf_v.at[sl],
            out_hbm.at[pl.ds(base + j * CH, CH), pl.ds(0, EMB)],
            osems.at[sl])

    for j in range(3):
        gat(j, j).start()

    def body(j, carry):
        sl = lax.rem(j, 4)
        gat(j, sl).wait()
        out(j, sl).start()

        @pl.when(j + 3 < NCH)
        def _():
            sl3 = lax.rem(j + 3, 4)

            @pl.when(j >= 1)
            def _():
                out(j - 1, sl3).wait()

            gat(j + 3, sl3).start()

        return carry

    lax.fori_loop(0, NCH, body, 0)
    for j in range(NCH - 4, NCH):
        out(j, j % 4).wait()


TILE = 4096


def _mlp_body(xs_ref, gcT_ref, w0a_ref, w0b_ref, b0_ref, w1_ref, b1_ref,
              wf_ref, bf_ref, eye_ref, xs_out, ex_ref):
    f32 = jnp.float32
    xr = xs_ref[:, :EMB]
    gcT = gcT_ref[...]
    gc = jax.lax.dot_general(gcT, eye_ref[...], (((0,), (0,)), ((), ())),
                             preferred_element_type=f32)
    xs_out[:, :EMB] = xr
    xs_out[:, EMB:] = gc
    h = jnp.dot(xr, w0a_ref[...], preferred_element_type=f32)
    h += jax.lax.dot_general(gcT, w0b_ref[...], (((0,), (0,)), ((), ())),
                             preferred_element_type=f32)
    h = jnp.maximum(h + b0_ref[...], 0.0)
    h = jnp.dot(h, w1_ref[...], preferred_element_type=f32)
    h = jnp.maximum(h + b1_ref[...], 0.0)
    logits = jnp.sum(h * wf_ref[...], axis=1) + bf_ref[0, 0]
    ex_ref[...] = jnp.exp(logits).reshape(TILE // CH, CH)


_mlp = pl.pallas_call(
    _mlp_body,
    grid=(N // TILE,),
    in_specs=[
        pl.BlockSpec((TILE, HID), lambda i: (i, 0)),
        pl.BlockSpec((EMB, TILE), lambda i: (0, i)),
        pl.BlockSpec((EMB, HID), lambda i: (0, 0)),
        pl.BlockSpec((EMB, HID), lambda i: (0, 0)),
        pl.BlockSpec((1, HID), lambda i: (0, 0)),
        pl.BlockSpec((HID, HID), lambda i: (0, 0)),
        pl.BlockSpec((1, HID), lambda i: (0, 0)),
        pl.BlockSpec((1, HID), lambda i: (0, 0)),
        pl.BlockSpec((1, 1), lambda i: (0, 0)),
        pl.BlockSpec((EMB, EMB), lambda i: (0, 0)),
    ],
    out_specs=[
        pl.BlockSpec((TILE, HID), lambda i: (i, 0)),
        pl.BlockSpec((TILE // CH, CH), lambda i: (i, 0)),
    ],
    out_shape=[
        jax.ShapeDtypeStruct((N, HID), jnp.float32),
        jax.ShapeDtypeStruct((NCHUNKS, CH), jnp.float32),
    ],
    input_output_aliases={0: 0},
    compiler_params=pltpu.CompilerParams(fuse_transposed_lhs_in_matmul=True),
)


@functools.partial(
    pl.kernel,
    mesh=_mesh,
    out_type=jax.ShapeDtypeStruct((NW, NCH, CH), jnp.float32),
    scratch_types=[
        pltpu.VMEM((CH_SC, CH), jnp.int32),
        pltpu.VMEM((CH_SC, CH), jnp.float32),
        pltpu.VMEM((NCH, CH), jnp.int32),
        pltpu.VMEM((NCH, CH), jnp.float32),
        pltpu.VMEM((NCH, CH), jnp.float32),
        pltpu.VMEM((B,), jnp.float32),
        pltpu.VMEM((B,), jnp.float32),
        pltpu.VMEM_SHARED((NS * B,), jnp.float32),
    ],
    compiler_params=pltpu.CompilerParams(needs_layout_passes=False),
)
def _seg_softmax(ex_sc_hbm, idx_sc_hbm, ex_hbm, idx_hbm, out_hbm,
                 idx_a, ex_a, idx_b, ex_b, out_v, table_v, stage_v, table_sh):
    c = lax.axis_index("c")
    s = lax.axis_index("s")
    wid = s * NC + c

    def zbody(i, carry):
        table_v[pl.ds(i * 16, 16)] = jnp.zeros((16,), jnp.float32)
        return carry
    lax.fori_loop(0, B // 16, zbody, 0)
    pltpu.sync_copy(table_v, table_sh.at[pl.ds(s * B, B)])

    pltpu.sync_copy(idx_sc_hbm.at[s], idx_a)
    pltpu.sync_copy(ex_sc_hbm.at[s], ex_a)
    off = (s * B).astype(jnp.int32)

    def obody(i, carry):
        r = i // (CH // 16)
        k = i % (CH // 16)
        sl = pl.ds(k * 16, 16)
        idx_a[r, sl] = idx_a[r, sl] + off
        return carry

    lax.fori_loop(0, CH_SC * (CH // 16), obody, 0)

    def sbody(j, carry):
        pltpu.sync_copy(ex_a.at[j], table_sh.at[idx_a.at[j]], add=True)
        return carry

    pltpu.sync_copy(idx_hbm.at[wid], idx_b)
    pltpu.sync_copy(ex_hbm.at[wid], ex_b)

    lax.fori_loop(0, CH_SC, sbody, 0)
    plsc.subcore_barrier()

    lo = idx_b[0, pl.ds(0, 16)][0]
    hi = idx_b[NCH - 1, pl.ds(CH - 16, 16)][15]
    BLK = 256
    kb0 = lo // BLK
    nb = hi // BLK - kb0 + 1

    def cpy_body(k, carry):
        base_b = (kb0 + k) * BLK
        pltpu.sync_copy(table_sh.at[pl.ds(base_b, BLK)],
                        table_v.at[pl.ds(base_b, BLK)])
        return carry

    lax.fori_loop(0, nb, cpy_body, 0)

    def cbody(r, carry):
        def kbody(k, carry2):
            base_b = (kb0 + k) * BLK
            pltpu.sync_copy(table_sh.at[pl.ds(r * B + base_b, BLK)],
                            stage_v.at[pl.ds(0, BLK)])

            def abody(i, carry3):
                dst = pl.ds(base_b + i * 16, 16)
                table_v[dst] = table_v[dst] + stage_v[pl.ds(i * 16, 16)]
                return carry3
            return lax.fori_loop(0, BLK // 16, abody, carry2)
        return lax.fori_loop(0, nb, kbody, carry)

    lax.fori_loop(1, NS, cbody, 0)


    def dbody(j, carry):
        def inner(k, carry2):
            idx16 = idx_b[j, pl.ds(k * 16, 16)]
            ex16 = ex_b[j, pl.ds(k * 16, 16)]
            den16 = plsc.load_gather(table_v, [idx16])
            out_v[j, pl.ds(k * 16, 16)] = ex16 / den16
            return carry2
        return lax.fori_loop(0, CH // 16, inner, carry)

    lax.fori_loop(0, NCH, dbody, 0)
    pltpu.sync_copy(out_v, out_hbm.at[wid])


def kernel(g_emb, g_candidates_emb, batch_idx, W0, b0, W1, b1, Wf, bf):
    idx3 = batch_idx.reshape(NW, NCH, CH)
    idx_sc = batch_idx.reshape(NS, CH_SC, CH)
    xs0 = _gather_rep(g_emb, idx3)
    x_states, ex = _mlp(
        xs0, g_candidates_emb.T,
        W0[:EMB], W0[EMB:],
        b0.reshape(1, HID), W1, b1.reshape(1, HID),
        Wf.reshape(1, HID), bf.reshape(1, 1),
        jnp.eye(EMB, dtype=jnp.float32),
    )
    probs = _seg_softmax(ex.reshape(NS, CH_SC, CH), idx_sc,
                         ex.reshape(NW, NCH, CH), idx3)
    return (g_emb, x_states, probs.reshape(N))


# speedup vs baseline: 2.3744x; 1.0550x over previous
import functools

import jax
import jax.numpy as jnp
from jax import lax
from jax.experimental import pallas as pl
from jax.experimental.pallas import tpu as pltpu
from jax.experimental.pallas import tpu_sc as plsc

B = 4096
N = 204800
EMB = 64
HID = 128

NC = 2
NS = 16
NW = NC * NS
ROWS_W = N // NW
CH = 128
NCH = ROWS_W // CH
NCHUNKS = N // CH
CH_SC = NCHUNKS // NS

_mesh = plsc.VectorSubcoreMesh(core_axis_name="c", subcore_axis_name="s")


@functools.partial(
    pl.kernel,
    mesh=_mesh,
    out_type=jax.ShapeDtypeStruct((N, HID), jnp.float32),
    scratch_types=[
        pltpu.VMEM((NCH // 2, 2 * CH), jnp.int32),
        pltpu.VMEM((4, 2 * CH, EMB), jnp.float32),
        pltpu.SemaphoreType.DMA((4,)),
        pltpu.SemaphoreType.DMA((4,)),
    ],
    compiler_params=pltpu.CompilerParams(use_tc_tiling_on_sc=False),
)
def _gather_rep(emb_hbm, idx_hbm, out_hbm, idx_v, buf_v, gsems, osems):
    c = lax.axis_index("c")
    s = lax.axis_index("s")
    wid = s * NC + c
    base = wid * ROWS_W
    pltpu.sync_copy(idx_hbm.at[wid], idx_v)
    NJ = NCH // 2

    def gat(j, sl):
        return pltpu.make_async_copy(
            emb_hbm.at[idx_v.at[j]], buf_v.at[sl], gsems.at[sl])

    def out(j, sl):
        return pltpu.make_async_copy(
            buf_v.at[sl],
            out_hbm.at[pl.ds(base + j * 2 * CH, 2 * CH), pl.ds(0, EMB)],
            osems.at[sl])

    for j in range(3):
        gat(j, j).start()

    def body(j, carry):
        sl = lax.rem(j, 4)
        gat(j, sl).wait()
        out(j, sl).start()

        @pl.when(j + 3 < NJ)
        def _():
            sl3 = lax.rem(j + 3, 4)

            @pl.when(j >= 1)
            def _():
                out(j - 1, sl3).wait()

            gat(j + 3, sl3).start()

        return carry

    lax.fori_loop(0, NJ, body, 0)
    for j in range(NJ - 4, NJ):
        out(j, j % 4).wait()


TILE = 4096


def _mlp_body(xs_ref, gcT_ref, w0a_ref, w0b_ref, b0_ref, w1_ref, b1_ref,
              wf_ref, bf_ref, eye_ref, xs_out, ex_ref):
    f32 = jnp.float32
    xr = xs_ref[:, :EMB]
    gcT = gcT_ref[...]
    gc = jax.lax.dot_general(gcT, eye_ref[...], (((0,), (0,)), ((), ())),
                             preferred_element_type=f32)
    xs_out[:, :EMB] = xr
    xs_out[:, EMB:] = gc
    h = jnp.dot(xr, w0a_ref[...], preferred_element_type=f32)
    h += jax.lax.dot_general(gcT, w0b_ref[...], (((0,), (0,)), ((), ())),
                             preferred_element_type=f32)
    h = jnp.maximum(h + b0_ref[...], 0.0)
    h = jnp.dot(h, w1_ref[...], preferred_element_type=f32)
    h = jnp.maximum(h + b1_ref[...], 0.0)
    logits = jnp.sum(h * wf_ref[...], axis=1) + bf_ref[0, 0]
    ex_ref[...] = jnp.exp(logits).reshape(TILE // CH, CH)


_mlp = pl.pallas_call(
    _mlp_body,
    grid=(N // TILE,),
    in_specs=[
        pl.BlockSpec((TILE, HID), lambda i: (i, 0)),
        pl.BlockSpec((EMB, TILE), lambda i: (0, i)),
        pl.BlockSpec((EMB, HID), lambda i: (0, 0)),
        pl.BlockSpec((EMB, HID), lambda i: (0, 0)),
        pl.BlockSpec((1, HID), lambda i: (0, 0)),
        pl.BlockSpec((HID, HID), lambda i: (0, 0)),
        pl.BlockSpec((1, HID), lambda i: (0, 0)),
        pl.BlockSpec((1, HID), lambda i: (0, 0)),
        pl.BlockSpec((1, 1), lambda i: (0, 0)),
        pl.BlockSpec((EMB, EMB), lambda i: (0, 0)),
    ],
    out_specs=[
        pl.BlockSpec((TILE, HID), lambda i: (i, 0)),
        pl.BlockSpec((TILE // CH, CH), lambda i: (i, 0)),
    ],
    out_shape=[
        jax.ShapeDtypeStruct((N, HID), jnp.float32),
        jax.ShapeDtypeStruct((NCHUNKS, CH), jnp.float32),
    ],
    input_output_aliases={0: 0},
    compiler_params=pltpu.CompilerParams(fuse_transposed_lhs_in_matmul=True),
)


@functools.partial(
    pl.kernel,
    mesh=_mesh,
    out_type=jax.ShapeDtypeStruct((NW, NCH, CH), jnp.float32),
    scratch_types=[
        pltpu.VMEM((CH_SC, CH), jnp.int32),
        pltpu.VMEM((CH_SC, CH), jnp.float32),
        pltpu.VMEM((NCH, CH), jnp.int32),
        pltpu.VMEM((NCH, CH), jnp.float32),
        pltpu.VMEM((NCH, CH), jnp.float32),
        pltpu.VMEM((B,), jnp.float32),
        pltpu.VMEM((B,), jnp.float32),
        pltpu.VMEM_SHARED((NS * B,), jnp.float32),
    ],
    compiler_params=pltpu.CompilerParams(needs_layout_passes=False),
)
def _seg_softmax(ex_sc_hbm, idx_sc_hbm, ex_hbm, idx_hbm, out_hbm,
                 idx_a, ex_a, idx_b, ex_b, out_v, table_v, stage_v, table_sh):
    c = lax.axis_index("c")
    s = lax.axis_index("s")
    wid = s * NC + c

    def zbody(i, carry):
        table_v[pl.ds(i * 16, 16)] = jnp.zeros((16,), jnp.float32)
        return carry
    lax.fori_loop(0, B // 16, zbody, 0)
    pltpu.sync_copy(table_v, table_sh.at[pl.ds(s * B, B)])

    pltpu.sync_copy(idx_sc_hbm.at[s], idx_a)
    pltpu.sync_copy(ex_sc_hbm.at[s], ex_a)
    off = (s * B).astype(jnp.int32)

    def obody(i, carry):
        r = i // (CH // 16)
        k = i % (CH // 16)
        sl = pl.ds(k * 16, 16)
        idx_a[r, sl] = idx_a[r, sl] + off
        return carry

    lax.fori_loop(0, CH_SC * (CH // 16), obody, 0)

    def sbody(j, carry):
        pltpu.sync_copy(ex_a.at[j], table_sh.at[idx_a.at[j]], add=True)
        return carry

    pltpu.sync_copy(idx_hbm.at[wid], idx_b)
    pltpu.sync_copy(ex_hbm.at[wid], ex_b)

    lax.fori_loop(0, CH_SC, sbody, 0)
    plsc.subcore_barrier()

    lo = idx_b[0, pl.ds(0, 16)][0]
    hi = idx_b[NCH - 1, pl.ds(CH - 16, 16)][15]
    BLK = 256
    kb0 = lo // BLK
    nb = hi // BLK - kb0 + 1

    def cpy_body(k, carry):
        base_b = (kb0 + k) * BLK
        pltpu.sync_copy(table_sh.at[pl.ds(base_b, BLK)],
                        table_v.at[pl.ds(base_b, BLK)])
        return carry

    lax.fori_loop(0, nb, cpy_body, 0)

    def cbody(r, carry):
        def kbody(k, carry2):
            base_b = (kb0 + k) * BLK
            pltpu.sync_copy(table_sh.at[pl.ds(r * B + base_b, BLK)],
                            stage_v.at[pl.ds(0, BLK)])

            def abody(i, carry3):
                dst = pl.ds(base_b + i * 16, 16)
                table_v[dst] = table_v[dst] + stage_v[pl.ds(i * 16, 16)]
                return carry3
            return lax.fori_loop(0, BLK // 16, abody, carry2)
        return lax.fori_loop(0, nb, kbody, carry)

    lax.fori_loop(1, NS, cbody, 0)


    def dbody(j, carry):
        def inner(k, carry2):
            idx16 = idx_b[j, pl.ds(k * 16, 16)]
            ex16 = ex_b[j, pl.ds(k * 16, 16)]
            den16 = plsc.load_gather(table_v, [idx16])
            out_v[j, pl.ds(k * 16, 16)] = ex16 / den16
            return carry2
        return lax.fori_loop(0, CH // 16, inner, carry)

    lax.fori_loop(0, NCH, dbody, 0)
    pltpu.sync_copy(out_v, out_hbm.at[wid])


def kernel(g_emb, g_candidates_emb, batch_idx, W0, b0, W1, b1, Wf, bf):
    idx3 = batch_idx.reshape(NW, NCH, CH)
    idx_sc = batch_idx.reshape(NS, CH_SC, CH)
    xs0 = _gather_rep(g_emb, batch_idx.reshape(NW, NCH // 2, 2 * CH))
    x_states, ex = _mlp(
        xs0, g_candidates_emb.T,
        W0[:EMB], W0[EMB:],
        b0.reshape(1, HID), W1, b1.reshape(1, HID),
        Wf.reshape(1, HID), bf.reshape(1, 1),
        jnp.eye(EMB, dtype=jnp.float32),
    )
    probs = _seg_softmax(ex.reshape(NS, CH_SC, CH), idx_sc,
                         ex.reshape(NW, NCH, CH), idx3)
    return (g_emb, x_states, probs.reshape(N))


# speedup vs baseline: 3.3863x; 1.4262x over previous
import functools

import jax
import jax.numpy as jnp
from jax import lax
from jax.experimental import pallas as pl
from jax.experimental.pallas import tpu as pltpu
from jax.experimental.pallas import tpu_sc as plsc

B = 4096
N = 204800
EMB = 64
HID = 128

NC = 2
NS = 16
NW = NC * NS
ROWS_W = N // NW
CH = 128
NCH = ROWS_W // CH
NCHUNKS = N // CH
CH_SC = NCHUNKS // NS

_mesh = plsc.VectorSubcoreMesh(core_axis_name="c", subcore_axis_name="s")


@functools.partial(
    pl.kernel,
    mesh=_mesh,
    out_type=jax.ShapeDtypeStruct((N, HID), jnp.float32),
    scratch_types=[
        pltpu.VMEM((NCH // 2, 2 * CH), jnp.int32),
        pltpu.VMEM((4, 2 * CH, EMB), jnp.float32),
        pltpu.SemaphoreType.DMA((4,)),
        pltpu.SemaphoreType.DMA((4,)),
        pltpu.VMEM_SHARED((B, EMB), jnp.float32),
    ],
    compiler_params=pltpu.CompilerParams(use_tc_tiling_on_sc=False),
)
def _gather_rep(emb_hbm, idx_hbm, out_hbm, idx_v, buf_v, gsems, osems,
                emb_sh):
    c = lax.axis_index("c")
    s = lax.axis_index("s")
    wid = s * NC + c
    base = wid * ROWS_W

    @pl.when(s == 0)
    def _():
        pltpu.sync_copy(emb_hbm, emb_sh)

    pltpu.sync_copy(idx_hbm.at[wid], idx_v)
    plsc.subcore_barrier()
    NJ = NCH // 2

    def gat(j, sl):
        return pltpu.make_async_copy(
            emb_sh.at[idx_v.at[j]], buf_v.at[sl], gsems.at[sl])

    def out(j, sl):
        return pltpu.make_async_copy(
            buf_v.at[sl],
            out_hbm.at[pl.ds(base + j * 2 * CH, 2 * CH), pl.ds(0, EMB)],
            osems.at[sl])

    for j in range(3):
        gat(j, j).start()

    def body(j, carry):
        sl = lax.rem(j, 4)
        gat(j, sl).wait()
        out(j, sl).start()

        @pl.when(j + 3 < NJ)
        def _():
            sl3 = lax.rem(j + 3, 4)

            @pl.when(j >= 1)
            def _():
                out(j - 1, sl3).wait()

            gat(j + 3, sl3).start()

        return carry

    lax.fori_loop(0, NJ, body, 0)
    for j in range(NJ - 4, NJ):
        out(j, j % 4).wait()


TILE = 4096


def _mlp_body(xs_ref, gcT_ref, w0a_ref, w0b_ref, b0_ref, w1_ref, b1_ref,
              wf_ref, bf_ref, eye_ref, xs_out, ex_ref):
    f32 = jnp.float32
    xr = xs_ref[:, :EMB]
    gcT = gcT_ref[...]
    gc = jax.lax.dot_general(gcT, eye_ref[...], (((0,), (0,)), ((), ())),
                             preferred_element_type=f32)
    xs_out[:, :EMB] = xr
    xs_out[:, EMB:] = gc
    h = jnp.dot(xr, w0a_ref[...], preferred_element_type=f32)
    h += jax.lax.dot_general(gcT, w0b_ref[...], (((0,), (0,)), ((), ())),
                             preferred_element_type=f32)
    h = jnp.maximum(h + b0_ref[...], 0.0)
    h = jnp.dot(h, w1_ref[...], preferred_element_type=f32)
    h = jnp.maximum(h + b1_ref[...], 0.0)
    logits = jnp.sum(h * wf_ref[...], axis=1) + bf_ref[0, 0]
    ex_ref[...] = jnp.exp(logits).reshape(TILE // CH, CH)


_mlp = pl.pallas_call(
    _mlp_body,
    grid=(N // TILE,),
    in_specs=[
        pl.BlockSpec((TILE, HID), lambda i: (i, 0)),
        pl.BlockSpec((EMB, TILE), lambda i: (0, i)),
        pl.BlockSpec((EMB, HID), lambda i: (0, 0)),
        pl.BlockSpec((EMB, HID), lambda i: (0, 0)),
        pl.BlockSpec((1, HID), lambda i: (0, 0)),
        pl.BlockSpec((HID, HID), lambda i: (0, 0)),
        pl.BlockSpec((1, HID), lambda i: (0, 0)),
        pl.BlockSpec((1, HID), lambda i: (0, 0)),
        pl.BlockSpec((1, 1), lambda i: (0, 0)),
        pl.BlockSpec((EMB, EMB), lambda i: (0, 0)),
    ],
    out_specs=[
        pl.BlockSpec((TILE, HID), lambda i: (i, 0)),
        pl.BlockSpec((TILE // CH, CH), lambda i: (i, 0)),
    ],
    out_shape=[
        jax.ShapeDtypeStruct((N, HID), jnp.float32),
        jax.ShapeDtypeStruct((NCHUNKS, CH), jnp.float32),
    ],
    input_output_aliases={0: 0},
    compiler_params=pltpu.CompilerParams(fuse_transposed_lhs_in_matmul=True),
)


@functools.partial(
    pl.kernel,
    mesh=_mesh,
    out_type=jax.ShapeDtypeStruct((NW, NCH, CH), jnp.float32),
    scratch_types=[
        pltpu.VMEM((CH_SC, CH), jnp.int32),
        pltpu.VMEM((CH_SC, CH), jnp.float32),
        pltpu.VMEM((NCH, CH), jnp.int32),
        pltpu.VMEM((NCH, CH), jnp.float32),
        pltpu.VMEM((NCH, CH), jnp.float32),
        pltpu.VMEM((B,), jnp.float32),
        pltpu.VMEM((B,), jnp.float32),
        pltpu.VMEM_SHARED((NS * B,), jnp.float32),
    ],
    compiler_params=pltpu.CompilerParams(needs_layout_passes=False),
)
def _seg_softmax(ex_sc_hbm, idx_sc_hbm, ex_hbm, idx_hbm, out_hbm,
                 idx_a, ex_a, idx_b, ex_b, out_v, table_v, stage_v, table_sh):
    c = lax.axis_index("c")
    s = lax.axis_index("s")
    wid = s * NC + c

    def zbody(i, carry):
        table_v[pl.ds(i * 16, 16)] = jnp.zeros((16,), jnp.float32)
        return carry
    lax.fori_loop(0, B // 16, zbody, 0)
    pltpu.sync_copy(table_v, table_sh.at[pl.ds(s * B, B)])

    pltpu.sync_copy(idx_sc_hbm.at[s], idx_a)
    pltpu.sync_copy(ex_sc_hbm.at[s], ex_a)
    off = (s * B).astype(jnp.int32)

    def obody(i, carry):
        r = i // (CH // 16)
        k = i % (CH // 16)
        sl = pl.ds(k * 16, 16)
        idx_a[r, sl] = idx_a[r, sl] + off
        return carry

    lax.fori_loop(0, CH_SC * (CH // 16), obody, 0)

    def sbody(j, carry):
        pltpu.sync_copy(ex_a.at[j], table_sh.at[idx_a.at[j]], add=True)
        return carry

    pltpu.sync_copy(idx_hbm.at[wid], idx_b)
    pltpu.sync_copy(ex_hbm.at[wid], ex_b)

    lax.fori_loop(0, CH_SC, sbody, 0)
    plsc.subcore_barrier()

    lo = idx_b[0, pl.ds(0, 16)][0]
    hi = idx_b[NCH - 1, pl.ds(CH - 16, 16)][15]
    BLK = 256
    kb0 = lo // BLK
    nb = hi // BLK - kb0 + 1

    def cpy_body(k, carry):
        base_b = (kb0 + k) * BLK
        pltpu.sync_copy(table_sh.at[pl.ds(base_b, BLK)],
                        table_v.at[pl.ds(base_b, BLK)])
        return carry

    lax.fori_loop(0, nb, cpy_body, 0)

    def cbody(r, carry):
        def kbody(k, carry2):
            base_b = (kb0 + k) * BLK
            pltpu.sync_copy(table_sh.at[pl.ds(r * B + base_b, BLK)],
                            stage_v.at[pl.ds(0, BLK)])

            def abody(i, carry3):
                dst = pl.ds(base_b + i * 16, 16)
                table_v[dst] = table_v[dst] + stage_v[pl.ds(i * 16, 16)]
                return carry3
            return lax.fori_loop(0, BLK // 16, abody, carry2)
        return lax.fori_loop(0, nb, kbody, carry)

    lax.fori_loop(1, NS, cbody, 0)


    def dbody(j, carry):
        def inner(k, carry2):
            idx16 = idx_b[j, pl.ds(k * 16, 16)]
            ex16 = ex_b[j, pl.ds(k * 16, 16)]
            den16 = plsc.load_gather(table_v, [idx16])
            out_v[j, pl.ds(k * 16, 16)] = ex16 / den16
            return carry2
        return lax.fori_loop(0, CH // 16, inner, carry)

    lax.fori_loop(0, NCH, dbody, 0)
    pltpu.sync_copy(out_v, out_hbm.at[wid])


def kernel(g_emb, g_candidates_emb, batch_idx, W0, b0, W1, b1, Wf, bf):
    idx3 = batch_idx.reshape(NW, NCH, CH)
    idx_sc = batch_idx.reshape(NS, CH_SC, CH)
    xs0 = _gather_rep(g_emb, batch_idx.reshape(NW, NCH // 2, 2 * CH))
    x_states, ex = _mlp(
        xs0, g_candidates_emb.T,
        W0[:EMB], W0[EMB:],
        b0.reshape(1, HID), W1, b1.reshape(1, HID),
        Wf.reshape(1, HID), bf.reshape(1, 1),
        jnp.eye(EMB, dtype=jnp.float32),
    )
    probs = _seg_softmax(ex.reshape(NS, CH_SC, CH), idx_sc,
                         ex.reshape(NW, NCH, CH), idx3)
    return (g_emb, x_states, probs.reshape(N))


# speedup vs baseline: 3.6856x; 1.0884x over previous
import functools

import jax
import jax.numpy as jnp
from jax import lax
from jax.experimental import pallas as pl
from jax.experimental.pallas import tpu as pltpu
from jax.experimental.pallas import tpu_sc as plsc

B = 4096
N = 204800
EMB = 64
HID = 128

NC = 2
NS = 16
NW = NC * NS
ROWS_W = N // NW
CH = 128
NCH = ROWS_W // CH
NCHUNKS = N // CH
CH_SC = NCHUNKS // NS

_mesh = plsc.VectorSubcoreMesh(core_axis_name="c", subcore_axis_name="s")


@functools.partial(
    pl.kernel,
    mesh=_mesh,
    out_type=jax.ShapeDtypeStruct((N, HID), jnp.float32),
    scratch_types=[
        pltpu.VMEM((NCH // 2, 2 * CH), jnp.int32),
        pltpu.VMEM((4, 2 * CH, EMB), jnp.float32),
        pltpu.SemaphoreType.DMA((4,)),
        pltpu.SemaphoreType.DMA((4,)),
        pltpu.VMEM_SHARED((B, EMB), jnp.float32),
    ],
    compiler_params=pltpu.CompilerParams(use_tc_tiling_on_sc=False),
)
def _gather_rep(emb_hbm, idx_hbm, out_hbm, idx_v, buf_v, gsems, osems,
                emb_sh):
    c = lax.axis_index("c")
    s = lax.axis_index("s")
    wid = s * NC + c
    base = wid * ROWS_W

    @pl.when(s == 0)
    def _():
        pltpu.sync_copy(emb_hbm, emb_sh)

    pltpu.sync_copy(idx_hbm.at[wid], idx_v)
    plsc.subcore_barrier()
    NJ = NCH // 2

    def gat(j, sl):
        return pltpu.make_async_copy(
            emb_sh.at[idx_v.at[j]], buf_v.at[sl], gsems.at[sl])

    def out(j, sl):
        return pltpu.make_async_copy(
            buf_v.at[sl],
            out_hbm.at[pl.ds(base + j * 2 * CH, 2 * CH), pl.ds(0, EMB)],
            osems.at[sl])

    for j in range(3):
        gat(j, j).start()

    def body(j, carry):
        sl = lax.rem(j, 4)
        gat(j, sl).wait()
        out(j, sl).start()

        @pl.when(j + 3 < NJ)
        def _():
            sl3 = lax.rem(j + 3, 4)

            @pl.when(j >= 1)
            def _():
                out(j - 1, sl3).wait()

            gat(j + 3, sl3).start()

        return carry

    lax.fori_loop(0, NJ, body, 0)
    for j in range(NJ - 4, NJ):
        out(j, j % 4).wait()


TILE = 8192


def _mlp_body(xs_ref, gcT_ref, w0a_ref, w0b_ref, b0_ref, w1_ref, b1_ref,
              wf_ref, bf_ref, eye_ref, xs_out, ex_ref):
    f32 = jnp.float32
    xr = xs_ref[:, :EMB]
    gcT = gcT_ref[...]
    gc = jax.lax.dot_general(gcT, eye_ref[...], (((0,), (0,)), ((), ())),
                             preferred_element_type=f32)
    xs_out[:, :EMB] = xr
    xs_out[:, EMB:] = gc
    h = jnp.dot(xr, w0a_ref[...], preferred_element_type=f32)
    h += jax.lax.dot_general(gcT, w0b_ref[...], (((0,), (0,)), ((), ())),
                             preferred_element_type=f32)
    h = jnp.maximum(h + b0_ref[...], 0.0)
    h = jnp.dot(h, w1_ref[...], preferred_element_type=f32)
    h = jnp.maximum(h + b1_ref[...], 0.0)
    logits = jnp.sum(h * wf_ref[...], axis=1) + bf_ref[0, 0]
    ex_ref[...] = jnp.exp(logits).reshape(TILE // CH, CH)


_mlp = pl.pallas_call(
    _mlp_body,
    grid=(N // TILE,),
    in_specs=[
        pl.BlockSpec((TILE, HID), lambda i: (i, 0)),
        pl.BlockSpec((EMB, TILE), lambda i: (0, i)),
        pl.BlockSpec((EMB, HID), lambda i: (0, 0)),
        pl.BlockSpec((EMB, HID), lambda i: (0, 0)),
        pl.BlockSpec((1, HID), lambda i: (0, 0)),
        pl.BlockSpec((HID, HID), lambda i: (0, 0)),
        pl.BlockSpec((1, HID), lambda i: (0, 0)),
        pl.BlockSpec((1, HID), lambda i: (0, 0)),
        pl.BlockSpec((1, 1), lambda i: (0, 0)),
        pl.BlockSpec((EMB, EMB), lambda i: (0, 0)),
    ],
    out_specs=[
        pl.BlockSpec((TILE, HID), lambda i: (i, 0)),
        pl.BlockSpec((TILE // CH, CH), lambda i: (i, 0)),
    ],
    out_shape=[
        jax.ShapeDtypeStruct((N, HID), jnp.float32),
        jax.ShapeDtypeStruct((NCHUNKS, CH), jnp.float32),
    ],
    input_output_aliases={0: 0},
    compiler_params=pltpu.CompilerParams(fuse_transposed_lhs_in_matmul=True),
)


@functools.partial(
    pl.kernel,
    mesh=_mesh,
    out_type=jax.ShapeDtypeStruct((NW, NCH, CH), jnp.float32),
    scratch_types=[
        pltpu.VMEM((CH_SC, CH), jnp.int32),
        pltpu.VMEM((CH_SC, CH), jnp.float32),
        pltpu.VMEM((NCH, CH), jnp.int32),
        pltpu.VMEM((NCH, CH), jnp.float32),
        pltpu.VMEM((NCH, CH), jnp.float32),
        pltpu.VMEM((B,), jnp.float32),
        pltpu.VMEM((B,), jnp.float32),
        pltpu.VMEM_SHARED((NS * B,), jnp.float32),
    ],
    compiler_params=pltpu.CompilerParams(needs_layout_passes=False),
)
def _seg_softmax(ex_sc_hbm, idx_sc_hbm, ex_hbm, idx_hbm, out_hbm,
                 idx_a, ex_a, idx_b, ex_b, out_v, table_v, stage_v, table_sh):
    c = lax.axis_index("c")
    s = lax.axis_index("s")
    wid = s * NC + c

    def zbody(i, carry):
        table_v[pl.ds(i * 16, 16)] = jnp.zeros((16,), jnp.float32)
        return carry
    lax.fori_loop(0, B // 16, zbody, 0)
    pltpu.sync_copy(table_v, table_sh.at[pl.ds(s * B, B)])

    pltpu.sync_copy(idx_sc_hbm.at[s], idx_a)
    pltpu.sync_copy(ex_sc_hbm.at[s], ex_a)
    off = (s * B).astype(jnp.int32)

    def obody(i, carry):
        r = i // (CH // 16)
        k = i % (CH // 16)
        sl = pl.ds(k * 16, 16)
        idx_a[r, sl] = idx_a[r, sl] + off
        return carry

    lax.fori_loop(0, CH_SC * (CH // 16), obody, 0)

    def sbody(j, carry):
        pltpu.sync_copy(ex_a.at[j], table_sh.at[idx_a.at[j]], add=True)
        return carry

    pltpu.sync_copy(idx_hbm.at[wid], idx_b)
    pltpu.sync_copy(ex_hbm.at[wid], ex_b)

    lax.fori_loop(0, CH_SC, sbody, 0)
    plsc.subcore_barrier()

    lo = idx_b[0, pl.ds(0, 16)][0]
    hi = idx_b[NCH - 1, pl.ds(CH - 16, 16)][15]
    BLK = 256
    kb0 = lo // BLK
    nb = hi // BLK - kb0 + 1

    def cpy_body(k, carry):
        base_b = (kb0 + k) * BLK
        pltpu.sync_copy(table_sh.at[pl.ds(base_b, BLK)],
                        table_v.at[pl.ds(base_b, BLK)])
        return carry

    lax.fori_loop(0, nb, cpy_body, 0)

    def cbody(r, carry):
        def kbody(k, carry2):
            base_b = (kb0 + k) * BLK
            pltpu.sync_copy(table_sh.at[pl.ds(r * B + base_b, BLK)],
                            stage_v.at[pl.ds(0, BLK)])

            def abody(i, carry3):
                dst = pl.ds(base_b + i * 16, 16)
                table_v[dst] = table_v[dst] + stage_v[pl.ds(i * 16, 16)]
                return carry3
            return lax.fori_loop(0, BLK // 16, abody, carry2)
        return lax.fori_loop(0, nb, kbody, carry)

    lax.fori_loop(1, NS, cbody, 0)


    def dbody(j, carry):
        def inner(k, carry2):
            idx16 = idx_b[j, pl.ds(k * 16, 16)]
            ex16 = ex_b[j, pl.ds(k * 16, 16)]
            den16 = plsc.load_gather(table_v, [idx16])
            out_v[j, pl.ds(k * 16, 16)] = ex16 / den16
            return carry2
        return lax.fori_loop(0, CH // 16, inner, carry)

    lax.fori_loop(0, NCH, dbody, 0)
    pltpu.sync_copy(out_v, out_hbm.at[wid])


def kernel(g_emb, g_candidates_emb, batch_idx, W0, b0, W1, b1, Wf, bf):
    idx3 = batch_idx.reshape(NW, NCH, CH)
    idx_sc = batch_idx.reshape(NS, CH_SC, CH)
    xs0 = _gather_rep(g_emb, batch_idx.reshape(NW, NCH // 2, 2 * CH))
    x_states, ex = _mlp(
        xs0, g_candidates_emb.T,
        W0[:EMB], W0[EMB:],
        b0.reshape(1, HID), W1, b1.reshape(1, HID),
        Wf.reshape(1, HID), bf.reshape(1, 1),
        jnp.eye(EMB, dtype=jnp.float32),
    )
    probs = _seg_softmax(ex.reshape(NS, CH_SC, CH), idx_sc,
                         ex.reshape(NW, NCH, CH), idx3)
    return (g_emb, x_states, probs.reshape(N))


# speedup vs baseline: 3.7028x; 1.0047x over previous
import functools

import jax
import jax.numpy as jnp
from jax import lax
from jax.experimental import pallas as pl
from jax.experimental.pallas import tpu as pltpu
from jax.experimental.pallas import tpu_sc as plsc

B = 4096
N = 204800
EMB = 64
HID = 128

NC = 2
NS = 16
NW = NC * NS
ROWS_W = N // NW
CH = 128
NCH = ROWS_W // CH
NCHUNKS = N // CH
CH_SC = NCHUNKS // NS

_mesh = plsc.VectorSubcoreMesh(core_axis_name="c", subcore_axis_name="s")


@functools.partial(
    pl.kernel,
    mesh=_mesh,
    out_type=jax.ShapeDtypeStruct((N, HID), jnp.float32),
    scratch_types=[
        pltpu.VMEM((NCH // 2, 2 * CH), jnp.int32),
        pltpu.VMEM((4, 2 * CH, EMB), jnp.float32),
        pltpu.SemaphoreType.DMA((4,)),
        pltpu.SemaphoreType.DMA((4,)),
        pltpu.VMEM_SHARED((B, EMB), jnp.float32),
    ],
    compiler_params=pltpu.CompilerParams(use_tc_tiling_on_sc=False),
)
def _gather_rep(emb_hbm, idx_hbm, out_hbm, idx_v, buf_v, gsems, osems,
                emb_sh):
    c = lax.axis_index("c")
    s = lax.axis_index("s")
    wid = s * NC + c
    base = wid * ROWS_W

    @pl.when(s == 0)
    def _():
        pltpu.sync_copy(emb_hbm, emb_sh)

    pltpu.sync_copy(idx_hbm.at[wid], idx_v)
    plsc.subcore_barrier()
    NJ = NCH // 2

    def gat(j, sl):
        return pltpu.make_async_copy(
            emb_sh.at[idx_v.at[j]], buf_v.at[sl], gsems.at[sl])

    def out(j, sl):
        return pltpu.make_async_copy(
            buf_v.at[sl],
            out_hbm.at[pl.ds(base + j * 2 * CH, 2 * CH), pl.ds(0, EMB)],
            osems.at[sl])

    for j in range(3):
        gat(j, j).start()

    def body(j, carry):
        sl = lax.rem(j, 4)
        gat(j, sl).wait()
        out(j, sl).start()

        @pl.when(j + 3 < NJ)
        def _():
            sl3 = lax.rem(j + 3, 4)

            @pl.when(j >= 1)
            def _():
                out(j - 1, sl3).wait()

            gat(j + 3, sl3).start()

        return carry

    lax.fori_loop(0, NJ, body, 0)
    for j in range(NJ - 4, NJ):
        out(j, j % 4).wait()


TILE = 10240


def _mlp_body(xs_ref, gcT_ref, w0a_ref, w0b_ref, b0_ref, w1_ref, b1_ref,
              wf_ref, bf_ref, eye_ref, xs_out, ex_ref):
    f32 = jnp.float32
    xr = xs_ref[:, :EMB]
    gcT = gcT_ref[...]
    gc = jax.lax.dot_general(gcT, eye_ref[...], (((0,), (0,)), ((), ())),
                             preferred_element_type=f32)
    xs_out[:, :EMB] = xr
    xs_out[:, EMB:] = gc
    h = jnp.dot(xr, w0a_ref[...], preferred_element_type=f32)
    h += jax.lax.dot_general(gcT, w0b_ref[...], (((0,), (0,)), ((), ())),
                             preferred_element_type=f32)
    h = jnp.maximum(h + b0_ref[...], 0.0)
    h = jnp.dot(h, w1_ref[...], preferred_element_type=f32)
    h = jnp.maximum(h + b1_ref[...], 0.0)
    logits = jnp.sum(h * wf_ref[...], axis=1) + bf_ref[0, 0]
    ex_ref[...] = jnp.exp(logits).reshape(TILE // CH, CH)


_mlp = pl.pallas_call(
    _mlp_body,
    grid=(N // TILE,),
    in_specs=[
        pl.BlockSpec((TILE, HID), lambda i: (i, 0)),
        pl.BlockSpec((EMB, TILE), lambda i: (0, i)),
        pl.BlockSpec((EMB, HID), lambda i: (0, 0)),
        pl.BlockSpec((EMB, HID), lambda i: (0, 0)),
        pl.BlockSpec((1, HID), lambda i: (0, 0)),
        pl.BlockSpec((HID, HID), lambda i: (0, 0)),
        pl.BlockSpec((1, HID), lambda i: (0, 0)),
        pl.BlockSpec((1, HID), lambda i: (0, 0)),
        pl.BlockSpec((1, 1), lambda i: (0, 0)),
        pl.BlockSpec((EMB, EMB), lambda i: (0, 0)),
    ],
    out_specs=[
        pl.BlockSpec((TILE, HID), lambda i: (i, 0)),
        pl.BlockSpec((TILE // CH, CH), lambda i: (i, 0)),
    ],
    out_shape=[
        jax.ShapeDtypeStruct((N, HID), jnp.float32),
        jax.ShapeDtypeStruct((NCHUNKS, CH), jnp.float32),
    ],
    input_output_aliases={0: 0},
    compiler_params=pltpu.CompilerParams(fuse_transposed_lhs_in_matmul=True),
)


@functools.partial(
    pl.kernel,
    mesh=_mesh,
    out_type=jax.ShapeDtypeStruct((NW, NCH, CH), jnp.float32),
    scratch_types=[
        pltpu.VMEM((CH_SC, CH), jnp.int32),
        pltpu.VMEM((CH_SC, CH), jnp.float32),
        pltpu.VMEM((NCH, CH), jnp.int32),
        pltpu.VMEM((NCH, CH), jnp.float32),
        pltpu.VMEM((NCH, CH), jnp.float32),
        pltpu.VMEM((B,), jnp.float32),
        pltpu.VMEM((B,), jnp.float32),
        pltpu.VMEM_SHARED((NS * B,), jnp.float32),
    ],
    compiler_params=pltpu.CompilerParams(needs_layout_passes=False),
)
def _seg_softmax(ex_sc_hbm, idx_sc_hbm, ex_hbm, idx_hbm, out_hbm,
                 idx_a, ex_a, idx_b, ex_b, out_v, table_v, stage_v, table_sh):
    c = lax.axis_index("c")
    s = lax.axis_index("s")
    wid = s * NC + c

    def zbody(i, carry):
        table_v[pl.ds(i * 16, 16)] = jnp.zeros((16,), jnp.float32)
        return carry
    lax.fori_loop(0, B // 16, zbody, 0)
    pltpu.sync_copy(table_v, table_sh.at[pl.ds(s * B, B)])

    pltpu.sync_copy(idx_sc_hbm.at[s], idx_a)
    pltpu.sync_copy(ex_sc_hbm.at[s], ex_a)
    off = (s * B).astype(jnp.int32)

    def obody(i, carry):
        r = i // (CH // 16)
        k = i % (CH // 16)
        sl = pl.ds(k * 16, 16)
        idx_a[r, sl] = idx_a[r, sl] + off
        return carry

    lax.fori_loop(0, CH_SC * (CH // 16), obody, 0)

    def sbody(j, carry):
        pltpu.sync_copy(ex_a.at[j], table_sh.at[idx_a.at[j]], add=True)
        return carry

    pltpu.sync_copy(idx_hbm.at[wid], idx_b)
    pltpu.sync_copy(ex_hbm.at[wid], ex_b)

    lax.fori_loop(0, CH_SC, sbody, 0)
    plsc.subcore_barrier()

    lo = idx_b[0, pl.ds(0, 16)][0]
    hi = idx_b[NCH - 1, pl.ds(CH - 16, 16)][15]
    BLK = 256
    kb0 = lo // BLK
    nb = hi // BLK - kb0 + 1

    def cpy_body(k, carry):
        base_b = (kb0 + k) * BLK
        pltpu.sync_copy(table_sh.at[pl.ds(base_b, BLK)],
                        table_v.at[pl.ds(base_b, BLK)])
        return carry

    lax.fori_loop(0, nb, cpy_body, 0)

    def cbody(r, carry):
        def kbody(k, carry2):
            base_b = (kb0 + k) * BLK
            pltpu.sync_copy(table_sh.at[pl.ds(r * B + base_b, BLK)],
                            stage_v.at[pl.ds(0, BLK)])

            def abody(i, carry3):
                dst = pl.ds(base_b + i * 16, 16)
                table_v[dst] = table_v[dst] + stage_v[pl.ds(i * 16, 16)]
                return carry3
            return lax.fori_loop(0, BLK // 16, abody, carry2)
        return lax.fori_loop(0, nb, kbody, carry)

    lax.fori_loop(1, NS, cbody, 0)


    def dbody(j, carry):
        def inner(k, carry2):
            idx16 = idx_b[j, pl.ds(k * 16, 16)]
            ex16 = ex_b[j, pl.ds(k * 16, 16)]
            den16 = plsc.load_gather(table_v, [idx16])
            out_v[j, pl.ds(k * 16, 16)] = ex16 / den16
            return carry2
        return lax.fori_loop(0, CH // 16, inner, carry)

    lax.fori_loop(0, NCH, dbody, 0)
    pltpu.sync_copy(out_v, out_hbm.at[wid])


def kernel(g_emb, g_candidates_emb, batch_idx, W0, b0, W1, b1, Wf, bf):
    idx3 = batch_idx.reshape(NW, NCH, CH)
    idx_sc = batch_idx.reshape(NS, CH_SC, CH)
    xs0 = _gather_rep(g_emb, batch_idx.reshape(NW, NCH // 2, 2 * CH))
    x_states, ex = _mlp(
        xs0, g_candidates_emb.T,
        W0[:EMB], W0[EMB:],
        b0.reshape(1, HID), W1, b1.reshape(1, HID),
        Wf.reshape(1, HID), bf.reshape(1, 1),
        jnp.eye(EMB, dtype=jnp.float32),
    )
    probs = _seg_softmax(ex.reshape(NS, CH_SC, CH), idx_sc,
                         ex.reshape(NW, NCH, CH), idx3)
    return (g_emb, x_states, probs.reshape(N))
